# Initial kernel scaffold; baseline (speedup 1.0000x reference)
import jax
import jax.numpy as jnp
from jax.experimental import pallas as pl


def kernel(rois, rcnn_conf, rcnn_deltas, input_image):
    raise NotImplementedError("write your pallas kernel here")


# speedup vs baseline: 26.0493x; 26.0493x over previous
import functools

import jax
import jax.numpy as jnp
from jax.experimental import pallas as pl
from jax.experimental.pallas import tpu as pltpu

_MAX_BOX = 100
_IOU_THRES = 0.5
_SCORE_THRES = 0.05
_NEG = -1e10


def _nms_body(H, W, B, C, N,
              s_in, ry1, rx1, ry2, rx2, t0, t1, t2, t3,
              oy1, ox1, oy2, ox2, osc, ocl, ond,
              sref, y1s, x1s, y2s, x2s, a2s):
    R, BPAD = s_in.shape
    f32 = jnp.float32

    w0 = rx2[...] - rx1[...] + 1.0
    h0 = ry2[...] - ry1[...] + 1.0
    x0 = rx1[...] + w0 / 2.0
    y0 = ry1[...] + h0 / 2.0
    cx = (t0[...] / 10.0) * w0 + x0
    cy = (t1[...] / 10.0) * h0 + y0
    ww = jnp.exp(t2[...] / 5.0) * w0
    hh = jnp.exp(t3[...] / 5.0) * h0
    xx1 = jnp.clip(cx - 0.5 * ww, 0.0, W - 1.0)
    yy1 = jnp.clip(cy - 0.5 * hh, 0.0, H - 1.0)
    xx2 = jnp.clip(cx + 0.5 * ww, 0.0, W - 1.0)
    yy2 = jnp.clip(cy + 0.5 * hh, 0.0, H - 1.0)
    y1s[...] = yy1
    x1s[...] = xx1
    y2s[...] = yy2
    x2s[...] = xx2
    a2s[...] = (yy2 - yy1) * (xx2 - xx1)

    lane = jax.lax.broadcasted_iota(jnp.int32, (R, BPAD), 1)
    s = s_in[...]
    sref[...] = jnp.where((lane < B) & (s > _SCORE_THRES), s, _NEG)

    lane128 = jax.lax.broadcasted_iota(jnp.int32, (R, 128), 1)

    def nms_step(k, carry):
        ss, sy1, sx1, sy2, sx2 = carry
        s = sref[...]
        best = jnp.max(s, axis=1, keepdims=True)
        idx = jnp.min(jnp.where(s == best, lane, BPAD), axis=1, keepdims=True)
        eq = lane == idx
        eqf = eq.astype(f32)
        y1 = y1s[...]
        x1 = x1s[...]
        y2 = y2s[...]
        x2 = x2s[...]
        by1 = jnp.sum(y1 * eqf, axis=1, keepdims=True)
        bx1 = jnp.sum(x1 * eqf, axis=1, keepdims=True)
        by2 = jnp.sum(y2 * eqf, axis=1, keepdims=True)
        bx2 = jnp.sum(x2 * eqf, axis=1, keepdims=True)
        yA = jnp.maximum(by1, y1)
        xA = jnp.maximum(bx1, x1)
        yB = jnp.minimum(by2, y2)
        xB = jnp.minimum(bx2, x2)
        inter = jnp.maximum(yB - yA, 0.0) * jnp.maximum(xB - xA, 0.0)
        a1 = (by2 - by1) * (bx2 - bx1)
        iou = inter / (a1 + a2s[...] - inter + 1e-8)
        sref[...] = jnp.where((iou > _IOU_THRES) | eq, _NEG, s)
        valid = best > _NEG * 0.5
        hit = lane128 == k
        ss = jnp.where(hit, jnp.where(valid, best, 0.0), ss)
        sy1 = jnp.where(hit, jnp.where(valid, by1, 0.0), sy1)
        sx1 = jnp.where(hit, jnp.where(valid, bx1, 0.0), sx1)
        sy2 = jnp.where(hit, jnp.where(valid, by2, 0.0), sy2)
        sx2 = jnp.where(hit, jnp.where(valid, bx2, 0.0), sx2)
        return ss, sy1, sx1, sy2, sx2

    init = (jnp.full((R, 128), -1.0, f32),
            jnp.zeros((R, 128), f32), jnp.zeros((R, 128), f32),
            jnp.zeros((R, 128), f32), jnp.zeros((R, 128), f32))
    ss, sy1, sx1, sy2, sx2 = jax.lax.fori_loop(0, _MAX_BOX, nms_step, init)

    row128 = jax.lax.broadcasted_iota(jnp.int32, (R, 128), 0)
    flatpos = row128 * 128 + lane128
    hit_iota = jax.lax.broadcasted_iota(jnp.int32, (1, 128), 1)

    per_image = []
    for n in range(N):
        ms0 = jnp.where((row128 >= n * C) & (row128 < (n + 1) * C), ss, -1.0)

        def merge_step(k, carry, n=n):
            ms, my1, mx1, my2, mx2, msc, mcl = carry
            m = jnp.max(jnp.max(ms, axis=1, keepdims=True), axis=0,
                        keepdims=True)
            cand = jnp.where(ms == m, flatpos, R * 128)
            pick = jnp.min(jnp.min(cand, axis=1, keepdims=True), axis=0,
                           keepdims=True)
            eq = flatpos == pick
            eqf = eq.astype(f32)

            def pick_val(a):
                return jnp.sum(jnp.sum(a * eqf, axis=1, keepdims=True),
                               axis=0, keepdims=True)

            by1 = pick_val(sy1)
            bx1 = pick_val(sx1)
            by2 = pick_val(sy2)
            bx2 = pick_val(sx2)
            cls = (pick // 128 - n * C).astype(f32)
            val = m > 0.0
            hit = hit_iota == k
            msc = jnp.where(hit, m, msc)
            my1 = jnp.where(hit, jnp.where(val, by1, 0.0), my1)
            mx1 = jnp.where(hit, jnp.where(val, bx1, 0.0), mx1)
            my2 = jnp.where(hit, jnp.where(val, by2, 0.0), my2)
            mx2 = jnp.where(hit, jnp.where(val, bx2, 0.0), mx2)
            mcl = jnp.where(hit, jnp.where(val, cls, 0.0), mcl)
            ms = jnp.where(eq, -2.0, ms)
            return ms, my1, mx1, my2, mx2, msc, mcl

        z = jnp.zeros((1, 128), f32)
        out = jax.lax.fori_loop(0, _MAX_BOX, merge_step,
                                (ms0, z, z, z, z, z, z))
        per_image.append(out[1:])

    oy1[...] = jnp.concatenate([per_image[0][0], per_image[1][0]], axis=0)
    ox1[...] = jnp.concatenate([per_image[0][1], per_image[1][1]], axis=0)
    oy2[...] = jnp.concatenate([per_image[0][2], per_image[1][2]], axis=0)
    ox2[...] = jnp.concatenate([per_image[0][3], per_image[1][3]], axis=0)
    osc[...] = jnp.concatenate([per_image[0][4], per_image[1][4]], axis=0)
    ocl[...] = jnp.concatenate([per_image[0][5], per_image[1][5]], axis=0)
    nds = []
    for n in range(N):
        msc_n = per_image[n][4]
        cnt = jnp.sum(jnp.sum((msc_n > 0.0).astype(jnp.int32), axis=1,
                              keepdims=True), axis=0, keepdims=True)
        nds.append(jnp.zeros((1, 128), jnp.int32) + cnt)
    ond[...] = jnp.concatenate(nds, axis=0)


def kernel(rois, rcnn_conf, rcnn_deltas, input_image):
    N, B = rois.shape[0], rois.shape[1]
    C = rcnn_conf.shape[2] - 1
    H = float(input_image.shape[2])
    W = float(input_image.shape[3])
    R = N * C
    BPAD = ((B + 127) // 128) * 128
    padw = BPAD - B

    scores = jnp.transpose(rcnn_conf[:, :, :C], (0, 2, 1)).reshape(R, B)
    dd = jnp.transpose(rcnn_deltas.reshape(N, B, C, 4), (0, 2, 3, 1))
    dd = dd.reshape(R, 4, B)
    rb = jnp.broadcast_to(rois[:, None, :, :], (N, C, B, 4)).reshape(R, B, 4)

    def p(x):
        return jnp.pad(x, ((0, 0), (0, padw)))

    args = [p(scores),
            p(rb[:, :, 0]), p(rb[:, :, 1]), p(rb[:, :, 2]), p(rb[:, :, 3]),
            p(dd[:, 0, :]), p(dd[:, 1, :]), p(dd[:, 2, :]), p(dd[:, 3, :])]

    f = jnp.float32
    out_shape = [jax.ShapeDtypeStruct((N, 128), f) for _ in range(6)]
    out_shape.append(jax.ShapeDtypeStruct((N, 128), jnp.int32))
    scratch = [pltpu.VMEM((R, BPAD), f) for _ in range(6)]

    body = functools.partial(_nms_body, H, W, B, C, N)
    oy1, ox1, oy2, ox2, osc, ocl, ond = pl.pallas_call(
        body,
        out_shape=out_shape,
        scratch_shapes=scratch,
    )(*args)

    nmsed_boxes = jnp.stack(
        [oy1[:, :_MAX_BOX], ox1[:, :_MAX_BOX],
         oy2[:, :_MAX_BOX], ox2[:, :_MAX_BOX]], axis=-1)
    nmsed_scores = osc[:, :_MAX_BOX]
    nmsed_classes = ocl[:, :_MAX_BOX]
    num_dets = ond[:, 0]
    return nmsed_boxes, nmsed_scores, nmsed_classes, num_dets, rois
